# baseline (speedup 1.0000x reference)
import functools

import jax
import jax.numpy as jnp
from jax.experimental import pallas as pl
from jax.experimental.pallas import tpu as pltpu

N_USERS = 4096
N_NEWS = 10000
D = 128
NEIGH = 20

USER_BLK = 256


def _user_stage_kernel(user_ref, inter_ref, agg_ref, out_ref):
    agg = agg_ref[...]
    ua = jnp.dot(inter_ref[...], agg, preferred_element_type=jnp.float32)
    logits = jnp.dot(user_ref[...], agg.T, preferred_element_type=jnp.float32)
    m = jnp.max(logits, axis=-1, keepdims=True)
    e = jnp.exp(logits - m)
    s = e / jnp.sum(e, axis=-1, keepdims=True)
    sa = jnp.dot(s, agg, preferred_element_type=jnp.float32)
    out_ref[...] = ua + sa * ua


def _user_stage(user_emb, interact_mat, news_agg):
    grid = (N_USERS // USER_BLK,)
    return pl.pallas_call(
        _user_stage_kernel,
        grid=grid,
        in_specs=[
            pl.BlockSpec((USER_BLK, D), lambda i: (i, 0)),
            pl.BlockSpec((USER_BLK, N_NEWS), lambda i: (i, 0)),
            pl.BlockSpec((N_NEWS, D), lambda i: (0, 0)),
        ],
        out_specs=pl.BlockSpec((USER_BLK, D), lambda i: (i, 0)),
        out_shape=jax.ShapeDtypeStruct((N_USERS, D), jnp.float32),
    )(user_emb, interact_mat, news_agg)


def _sim_hrt(emb_head, emb_tail, relation_emb):
    tail_relation_emb = emb_tail * relation_emb
    head_relation_emb = emb_head[:, None, :] * relation_emb
    head_relation_emb = jnp.linalg.norm(head_relation_emb, ord=2, axis=1, keepdims=True)
    att = jnp.matmul(tail_relation_emb, jnp.swapaxes(head_relation_emb, -2, -1))
    return jnp.squeeze(att, -1) ** 2


def _aggregate(head_emb, entity_emb, relation_emb, nbr_entities, nbr_relations):
    tails = jnp.take(entity_emb, nbr_entities, axis=0)
    rels = jnp.take(relation_emb, nbr_relations, axis=0)
    w = jax.nn.softmax(_sim_hrt(head_emb, tails, rels), axis=-1)
    return jnp.sum(w[..., None] * tails, axis=1)


def kernel(user_emb, news_embeding, entity_emb, relation_emb, interact_mat,
           news_entities, news_relations, neigh_entities, neigh_relations):
    news_agg = _aggregate(news_embeding, entity_emb, relation_emb,
                          news_entities, news_relations)
    entity_agg = _aggregate(entity_emb, entity_emb, relation_emb,
                            neigh_entities, neigh_relations)
    user_agg = _user_stage(user_emb, interact_mat, news_agg)
    return (news_agg, entity_agg, user_agg)
